# Initial kernel scaffold; baseline (speedup 1.0000x reference)
import jax
import jax.numpy as jnp
from jax.experimental import pallas as pl


def kernel(zi, zj, zn, pos_edge_index, neg_edge_index, cuda):
    raise NotImplementedError("write your pallas kernel here")


# speedup vs baseline: 3.3829x; 3.3829x over previous
import functools

import jax
import jax.numpy as jnp
from jax import lax
from jax.experimental import pallas as pl
from jax.experimental.pallas import tpu as pltpu
from jax.experimental.pallas import tpu_sc as plsc

N_NODES = 10000
D_FEAT = 128
E_HALF = 320000
E_TOTAL = 2 * E_HALF

NC = 2
NS = 16
L = 16
NW = NC * NS

CH = 128
CHUNKS_PER_W = 157
E_PAD = NW * CHUNKS_PER_W * CH


def _sc_scores(zi, table, src_idx, dst_idx):
    mesh = plsc.VectorSubcoreMesh(core_axis_name="c", subcore_axis_name="s")

    @functools.partial(
        pl.kernel,
        mesh=mesh,
        compiler_params=pltpu.CompilerParams(needs_layout_passes=False),
        out_type=jax.ShapeDtypeStruct((E_PAD,), jnp.float32),
        scratch_types=[
            pltpu.VMEM((CH,), jnp.int32),
            pltpu.VMEM((CH,), jnp.int32),
            pltpu.VMEM((CH, D_FEAT), jnp.float32),
            pltpu.VMEM((CH, D_FEAT), jnp.float32),
            pltpu.VMEM((L * L,), jnp.float32),
            pltpu.VMEM((CH,), jnp.float32),
            pltpu.SemaphoreType.DMA,
            pltpu.SemaphoreType.DMA,
        ],
    )
    def k(zi_hbm, tab_hbm, si_hbm, di_hbm, out_hbm,
          si_v, di_v, a_v, b_v, tmp_v, o_v, sem_a, sem_b):
        wid = lax.axis_index("s") * NC + lax.axis_index("c")
        base = wid * (CHUNKS_PER_W * CH)
        lane = lax.iota(jnp.int32, 16) * L

        @pl.loop(0, CHUNKS_PER_W)
        def _(t):
            off = base + t * CH
            pltpu.sync_copy(si_hbm.at[pl.ds(off, CH)], si_v)
            pltpu.sync_copy(di_hbm.at[pl.ds(off, CH)], di_v)
            cp_a = pltpu.async_copy(zi_hbm.at[si_v], a_v, sem_a)
            cp_b = pltpu.async_copy(tab_hbm.at[di_v], b_v, sem_b)
            cp_a.wait()
            cp_b.wait()

            @pl.loop(0, CH // L)
            def _(g):
                e0 = g * L
                for i in range(L):
                    e = e0 + i
                    acc = a_v[e, pl.ds(0, L)] * b_v[e, pl.ds(0, L)]
                    for kk in range(1, D_FEAT // L):
                        acc = acc + (a_v[e, pl.ds(kk * L, L)]
                                     * b_v[e, pl.ds(kk * L, L)])
                    tmp_v[pl.ds(i * L, L)] = acc
                s = plsc.load_gather(tmp_v, [lane])
                for dcol in range(1, L):
                    s = s + plsc.load_gather(tmp_v, [lane + dcol])
                o_v[pl.ds(e0, L)] = s

            pltpu.sync_copy(o_v, out_hbm.at[pl.ds(off, CH)])

    return k(zi, table, src_idx, dst_idx)


def _tc_loss(scores):
    rows = E_PAD // 1024

    def body(s_ref, o_ref):
        s = s_ref[...]
        r = lax.broadcasted_iota(jnp.int32, (rows, 1024), 0)
        c = lax.broadcasted_iota(jnp.int32, (rows, 1024), 1)
        li = r * 1024 + c
        x = jnp.where(li < E_HALF, -s, s)
        sp = jnp.maximum(x, 0.0) + jnp.log1p(jnp.exp(-jnp.abs(x)))
        sp = jnp.where(li < E_TOTAL, sp, 0.0)
        o_ref[...] = (jnp.sum(sp) / E_TOTAL).reshape(1, 1)

    out = pl.pallas_call(
        body,
        out_shape=jax.ShapeDtypeStruct((1, 1), jnp.float32),
    )(scores.reshape(rows, 1024))
    return out[0, 0]


def kernel(zi, zj, zn, pos_edge_index, neg_edge_index, cuda):
    src = jnp.concatenate(
        [pos_edge_index[0], neg_edge_index[0]]).astype(jnp.int32)
    dst = jnp.concatenate(
        [pos_edge_index[1], neg_edge_index[1] + N_NODES]).astype(jnp.int32)
    pad = E_PAD - E_TOTAL
    src = jnp.concatenate([src, jnp.zeros((pad,), jnp.int32)])
    dst = jnp.concatenate([dst, jnp.zeros((pad,), jnp.int32)])
    table = jnp.concatenate([zj, zn], axis=0)
    scores = _sc_scores(zi, table, src, dst)
    return _tc_loss(scores)


# speedup vs baseline: 4.4543x; 1.3167x over previous
import functools

import jax
import jax.numpy as jnp
from jax import lax
from jax.experimental import pallas as pl
from jax.experimental.pallas import tpu as pltpu
from jax.experimental.pallas import tpu_sc as plsc

N_NODES = 10000
D_FEAT = 128
E_HALF = 320000
E_TOTAL = 2 * E_HALF

NC = 2
NS = 16
L = 16
NW = NC * NS

CH = 128
CHUNKS_PER_W = 158
PW = CHUNKS_PER_W * CH
E_PAD = NW * PW


def _sc_scores(zi, table, src_idx, dst_idx):
    mesh = plsc.VectorSubcoreMesh(core_axis_name="c", subcore_axis_name="s")

    @functools.partial(
        pl.kernel,
        mesh=mesh,
        compiler_params=pltpu.CompilerParams(needs_layout_passes=False),
        out_type=jax.ShapeDtypeStruct((E_PAD,), jnp.float32),
        scratch_types=[
            pltpu.VMEM((PW,), jnp.int32),
            pltpu.VMEM((PW,), jnp.int32),
            pltpu.VMEM((CH, D_FEAT), jnp.float32),
            pltpu.VMEM((CH, D_FEAT), jnp.float32),
            pltpu.VMEM((CH, D_FEAT), jnp.float32),
            pltpu.VMEM((CH, D_FEAT), jnp.float32),
            pltpu.VMEM((L * L,), jnp.float32),
            pltpu.VMEM((PW,), jnp.float32),
            pltpu.SemaphoreType.DMA,
            pltpu.SemaphoreType.DMA,
        ],
    )
    def k(zi_hbm, tab_hbm, si_hbm, di_hbm, out_hbm,
          si_all, di_all, a0_v, b0_v, a1_v, b1_v, tmp_v, o_all, sem0, sem1):
        wid = lax.axis_index("s") * NC + lax.axis_index("c")
        base = wid * PW
        lane = lax.iota(jnp.int32, 16) * L

        pltpu.sync_copy(si_hbm.at[pl.ds(base, PW)], si_all)
        pltpu.sync_copy(di_hbm.at[pl.ds(base, PW)], di_all)

        def issue(t, a_v, b_v, sem):
            pltpu.async_copy(zi_hbm.at[si_all.at[pl.ds(t * CH, CH)]], a_v, sem)
            pltpu.async_copy(tab_hbm.at[di_all.at[pl.ds(t * CH, CH)]], b_v, sem)

        def drain(a_v, b_v, sem):
            pltpu.make_async_copy(
                zi_hbm.at[si_all.at[pl.ds(0, CH)]], a_v, sem).wait()
            pltpu.make_async_copy(
                tab_hbm.at[di_all.at[pl.ds(0, CH)]], b_v, sem).wait()

        def compute(t, a_v, b_v):
            @pl.loop(0, CH // L)
            def _(g):
                e0 = g * L
                for i in range(L):
                    e = e0 + i
                    acc = a_v[e, pl.ds(0, L)] * b_v[e, pl.ds(0, L)]
                    for kk in range(1, D_FEAT // L):
                        acc = acc + (a_v[e, pl.ds(kk * L, L)]
                                     * b_v[e, pl.ds(kk * L, L)])
                    tmp_v[pl.ds(i * L, L)] = acc
                s = plsc.load_gather(tmp_v, [lane])
                for dcol in range(1, L):
                    s = s + plsc.load_gather(tmp_v, [lane + dcol])
                o_all[pl.ds(t * CH + e0, L)] = s

        issue(0, a0_v, b0_v, sem0)
        issue(1, a1_v, b1_v, sem1)

        @pl.loop(0, CHUNKS_PER_W, step=2)
        def _(t):
            drain(a0_v, b0_v, sem0)
            compute(t, a0_v, b0_v)

            @pl.when(t + 2 < CHUNKS_PER_W)
            def _():
                issue(t + 2, a0_v, b0_v, sem0)

            drain(a1_v, b1_v, sem1)
            compute(t + 1, a1_v, b1_v)

            @pl.when(t + 3 < CHUNKS_PER_W)
            def _():
                issue(t + 3, a1_v, b1_v, sem1)

        pltpu.sync_copy(o_all, out_hbm.at[pl.ds(base, PW)])

    return k(zi, table, src_idx, dst_idx)


def _tc_loss(scores):
    rows = E_PAD // 1024

    def body(s_ref, o_ref):
        s = s_ref[...]
        r = lax.broadcasted_iota(jnp.int32, (rows, 1024), 0)
        c = lax.broadcasted_iota(jnp.int32, (rows, 1024), 1)
        li = r * 1024 + c
        x = jnp.where(li < E_HALF, -s, s)
        sp = jnp.maximum(x, 0.0) + jnp.log1p(jnp.exp(-jnp.abs(x)))
        sp = jnp.where(li < E_TOTAL, sp, 0.0)
        o_ref[...] = (jnp.sum(sp) / E_TOTAL).reshape(1, 1)

    out = pl.pallas_call(
        body,
        out_shape=jax.ShapeDtypeStruct((1, 1), jnp.float32),
    )(scores.reshape(rows, 1024))
    return out[0, 0]


def kernel(zi, zj, zn, pos_edge_index, neg_edge_index, cuda):
    src = jnp.concatenate(
        [pos_edge_index[0], neg_edge_index[0]]).astype(jnp.int32)
    dst = jnp.concatenate(
        [pos_edge_index[1], neg_edge_index[1] + N_NODES]).astype(jnp.int32)
    pad = E_PAD - E_TOTAL
    src = jnp.concatenate([src, jnp.zeros((pad,), jnp.int32)])
    dst = jnp.concatenate([dst, jnp.zeros((pad,), jnp.int32)])
    table = jnp.concatenate([zj, zn], axis=0)
    scores = _sc_scores(zi, table, src, dst)
    return _tc_loss(scores)


# speedup vs baseline: 5.5581x; 1.2478x over previous
import functools

import jax
import jax.numpy as jnp
from jax import lax
from jax.experimental import pallas as pl
from jax.experimental.pallas import tpu as pltpu
from jax.experimental.pallas import tpu_sc as plsc

N_NODES = 10000
D_FEAT = 128
E_HALF = 320000
E_TOTAL = 2 * E_HALF

NC = 2
NS = 16
L = 16
NW = NC * NS

CH = 128
CHUNKS_PER_W = 158
PW = CHUNKS_PER_W * CH
E_PAD = NW * PW


def _sc_scores(zi, table, src_idx, dst_idx):
    mesh = plsc.VectorSubcoreMesh(core_axis_name="c", subcore_axis_name="s")

    @functools.partial(
        pl.kernel,
        mesh=mesh,
        compiler_params=pltpu.CompilerParams(
            needs_layout_passes=False, use_tc_tiling_on_sc=False),
        out_type=jax.ShapeDtypeStruct((E_PAD,), jnp.float32),
        scratch_types=[
            pltpu.VMEM((PW,), jnp.int32),
            pltpu.VMEM((PW,), jnp.int32),
            pltpu.VMEM((CH, D_FEAT // 2), jnp.int32),
            pltpu.VMEM((CH, D_FEAT // 2), jnp.int32),
            pltpu.VMEM((CH, D_FEAT // 2), jnp.int32),
            pltpu.VMEM((CH, D_FEAT // 2), jnp.int32),
            pltpu.VMEM((L * L,), jnp.float32),
            pltpu.VMEM((PW,), jnp.float32),
            pltpu.SemaphoreType.DMA,
            pltpu.SemaphoreType.DMA,
        ],
    )
    def k(zi_hbm, tab_hbm, si_hbm, di_hbm, out_hbm,
          si_all, di_all, a0_v, b0_v, a1_v, b1_v, tmp_v, o_all, sem0, sem1):
        wid = lax.axis_index("s") * NC + lax.axis_index("c")
        base = wid * PW
        lane = lax.iota(jnp.int32, 16) * L

        pltpu.sync_copy(si_hbm.at[pl.ds(base, PW)], si_all)
        pltpu.sync_copy(di_hbm.at[pl.ds(base, PW)], di_all)

        def issue(t, a_v, b_v, sem):
            pltpu.async_copy(zi_hbm.at[si_all.at[pl.ds(t * CH, CH)]], a_v, sem)
            pltpu.async_copy(tab_hbm.at[di_all.at[pl.ds(t * CH, CH)]], b_v, sem)

        def drain(a_v, b_v, sem):
            pltpu.make_async_copy(
                zi_hbm.at[si_all.at[pl.ds(0, CH)]], a_v, sem).wait()
            pltpu.make_async_copy(
                tab_hbm.at[di_all.at[pl.ds(0, CH)]], b_v, sem).wait()

        def compute(t, a_v, b_v):
            @pl.loop(0, CH // L)
            def _(g):
                e0 = g * L
                for i in range(L):
                    e = e0 + i

                    def chunk(kk):
                        aa = plsc.bitcast(a_v[e, pl.ds(kk * L, L)], jnp.bfloat16)
                        bb = plsc.bitcast(b_v[e, pl.ds(kk * L, L)], jnp.bfloat16)
                        return aa * bb

                    acc = chunk(0)
                    for kk in range(1, D_FEAT // (2 * L)):
                        acc = acc + chunk(kk)
                    lo, hi = plsc.unpack(
                        acc, format=plsc.PackFormat.INTERLEAVED,
                        preferred_element_type=jnp.float32)
                    tmp_v[pl.ds(i * L, L)] = lo + hi
                s = plsc.load_gather(tmp_v, [lane])
                for dcol in range(1, L):
                    s = s + plsc.load_gather(tmp_v, [lane + dcol])
                o_all[pl.ds(t * CH + e0, L)] = s

        issue(0, a0_v, b0_v, sem0)
        issue(1, a1_v, b1_v, sem1)

        @pl.loop(0, CHUNKS_PER_W, step=2)
        def _(t):
            drain(a0_v, b0_v, sem0)
            compute(t, a0_v, b0_v)

            @pl.when(t + 2 < CHUNKS_PER_W)
            def _():
                issue(t + 2, a0_v, b0_v, sem0)

            drain(a1_v, b1_v, sem1)
            compute(t + 1, a1_v, b1_v)

            @pl.when(t + 3 < CHUNKS_PER_W)
            def _():
                issue(t + 3, a1_v, b1_v, sem1)

        pltpu.sync_copy(o_all, out_hbm.at[pl.ds(base, PW)])

    return k(zi, table, src_idx, dst_idx)


def _tc_loss(scores):
    rows = E_PAD // 1024

    def body(s_ref, o_ref):
        s = s_ref[...]
        r = lax.broadcasted_iota(jnp.int32, (rows, 1024), 0)
        c = lax.broadcasted_iota(jnp.int32, (rows, 1024), 1)
        li = r * 1024 + c
        x = jnp.where(li < E_HALF, -s, s)
        sp = jnp.maximum(x, 0.0) + jnp.log1p(jnp.exp(-jnp.abs(x)))
        sp = jnp.where(li < E_TOTAL, sp, 0.0)
        o_ref[...] = (jnp.sum(sp) / E_TOTAL).reshape(1, 1)

    out = pl.pallas_call(
        body,
        out_shape=jax.ShapeDtypeStruct((1, 1), jnp.float32),
    )(scores.reshape(rows, 1024))
    return out[0, 0]


def kernel(zi, zj, zn, pos_edge_index, neg_edge_index, cuda):
    src = jnp.concatenate(
        [pos_edge_index[0], neg_edge_index[0]]).astype(jnp.int32)
    dst = jnp.concatenate(
        [pos_edge_index[1], neg_edge_index[1] + N_NODES]).astype(jnp.int32)
    pad = E_PAD - E_TOTAL
    src = jnp.concatenate([src, jnp.zeros((pad,), jnp.int32)])
    dst = jnp.concatenate([dst, jnp.zeros((pad,), jnp.int32)])
    def pack_bf16(x):
        xb = x.astype(jnp.bfloat16)
        return jax.lax.bitcast_convert_type(
            xb.reshape(x.shape[0], D_FEAT // 2, 2), jnp.int32)

    table = pack_bf16(jnp.concatenate([zj, zn], axis=0))
    scores = _sc_scores(pack_bf16(zi), table, src, dst)
    return _tc_loss(scores)
